# baseline (speedup 1.0000x reference)
import functools

import jax
import jax.numpy as jnp
from jax import lax
from jax.experimental import pallas as pl
from jax.experimental.pallas import tpu as pltpu


def _round_up(n, m):
    return ((n + m - 1) // m) * m


def _lstm_tp_kernel(x_ref, w_ref, fcw_ref, out_ref, *, gb, nc):
    n_outer, U, L = x_ref.shape
    LH = L // nc

    row_iota = lax.broadcasted_iota(jnp.int32, (8, LH), 0)
    is_one_row = row_iota == 1

    def make_ext(x_row):
        xb = jnp.broadcast_to(x_row, (8, LH))
        return jnp.where(is_one_row, 1.0, xb)

    def cell(g, c):
        th = jnp.tanh(g[: 3 * gb])
        tg = jnp.tanh(g[3 * gb:])
        ti = th[:gb]
        tf = th[gb: 2 * gb]
        to = th[2 * gb: 3 * gb]
        c_new = 0.5 * ((c + tf * c) + (tg + ti * tg))
        t2 = jnp.tanh(c_new)
        h2 = t2 + to * t2
        return h2, c_new

    def outer(j, carry):
        hs, cs = carry
        x_u = x_ref[j]
        for k in range(U):
            gs = []
            for n in range(nc):
                x_row = x_u[k: k + 1, n * LH: (n + 1) * LH]
                rhs = jnp.concatenate(
                    [hs[n], make_ext(x_row)], axis=0).astype(jnp.bfloat16)
                gs.append(jnp.dot(w_ref[...], rhs,
                                  preferred_element_type=jnp.float32))
            new_hs, new_cs = [], []
            for n in range(nc):
                h_new, c_new = cell(gs[n], cs[n])
                new_hs.append(h_new)
                new_cs.append(c_new)
            hs, cs = tuple(new_hs), tuple(new_cs)
        return (hs, cs)

    h0 = tuple(jnp.zeros((gb, LH), jnp.float32) for _ in range(nc))
    c0 = tuple(jnp.zeros((gb, LH), jnp.float32) for _ in range(nc))
    hs, cs = lax.fori_loop(0, n_outer, outer, (h0, c0), unroll=1)

    zero_row = jnp.zeros((1, LH), jnp.float32)
    for n in range(nc):
        rhs_last = jnp.concatenate(
            [hs[n], make_ext(zero_row)], axis=0).astype(jnp.bfloat16)
        out_ref[:, n * LH: (n + 1) * LH] = jnp.dot(
            fcw_ref[...], rhs_last, preferred_element_type=jnp.float32)


def kernel(x, w_ih, w_hh, b, fc_w, fc_b):
    B, T, I = x.shape
    H = w_hh.shape[-1]
    F = fc_w.shape[-1]

    GB = _round_up(H, 8)
    KD = GB + 8
    FP = _round_up(F, 8)

    NC = 8
    LH = 256
    while NC > 1 and B % (LH * NC):
        NC //= 2
    while B % LH:
        LH //= 2
    LANES = LH * NC
    assert B % LANES == 0

    U = 8
    while T % U:
        U //= 2

    order = ((0, 0.5), (1, 0.5), (3, 0.5), (2, 1.0))
    blocks = []
    for k, scale in order:
        blk = jnp.zeros((GB, KD), jnp.float32)
        blk = blk.at[:H, :H].set(w_hh[k].T * (scale * 0.5))
        blk = blk.at[:H, GB].set(w_ih[k][0] * scale)
        blk = blk.at[:H, GB + 1].set(b[k][0] * scale)
        blocks.append(blk)
    w_pack = jnp.concatenate(blocks, axis=0).astype(jnp.bfloat16)

    fcw_pack = jnp.zeros((FP, KD), jnp.float32)
    fcw_pack = fcw_pack.at[:F, :H].set(fc_w.T * 0.5)
    fcw_pack = fcw_pack.at[:F, GB + 1].set(fc_b[0])
    fcw_pack = fcw_pack.astype(jnp.bfloat16)

    x_tm = jnp.transpose(x[:, :, 0]).reshape(T // U, U, B)

    grid = (B // LANES,)
    out_t = pl.pallas_call(
        functools.partial(_lstm_tp_kernel, gb=GB, nc=NC),
        out_shape=jax.ShapeDtypeStruct((FP, B), jnp.float32),
        grid=grid,
        in_specs=[
            pl.BlockSpec((T // U, U, LANES), lambda i: (0, 0, i)),
            pl.BlockSpec((4 * GB, KD), lambda i: (0, 0)),
            pl.BlockSpec((FP, KD), lambda i: (0, 0)),
        ],
        out_specs=pl.BlockSpec((FP, LANES), lambda i: (0, i)),
        compiler_params=pltpu.CompilerParams(
            dimension_semantics=("parallel",)),
    )(x_tm, w_pack, fcw_pack)

    return jnp.transpose(out_t)[:, :F]


# speedup vs baseline: 1.0794x; 1.0794x over previous
import functools

import jax
import jax.numpy as jnp
from jax import lax
from jax.experimental import pallas as pl
from jax.experimental.pallas import tpu as pltpu


def _round_up(n, m):
    return ((n + m - 1) // m) * m


def _lstm_tp_kernel(x_ref, w_ref, fcw_ref, out_ref, *, gb, nc):
    n_outer, U, L = x_ref.shape
    LH = L // nc

    row_iota = lax.broadcasted_iota(jnp.int32, (8, LH), 0)
    is_one_row = row_iota == 1

    def make_ext(x_row):
        xb = jnp.broadcast_to(x_row, (8, LH))
        return jnp.where(is_one_row, 1.0, xb)

    def cell(g, c):
        th = jnp.tanh(g[:144])
        tg = jnp.tanh(g[144:192])
        tl = jnp.tanh(g[192:200])
        ti = th[:48]
        tf = th[48:96]
        to = th[96:144]
        c_main = c[:48]
        c_tail = c[48:56]
        cm = 0.5 * ((c_main + tf * c_main) + (tg + ti * tg))
        t2m = jnp.tanh(cm)
        h2m = t2m + to * t2m
        r2 = pltpu.roll(tl, 6, 0)
        r4 = pltpu.roll(tl, 4, 0)
        r6 = pltpu.roll(tl, 2, 0)
        ctl = 0.5 * ((c_tail + r2 * c_tail) + (r4 + tl * r4))
        t2t = jnp.tanh(ctl)
        h2t = t2t + r6 * t2t
        h2 = jnp.concatenate([h2m, h2t], axis=0)
        c_new = jnp.concatenate([cm, ctl], axis=0)
        return h2, c_new

    def outer(j, carry):
        hs, cs = carry
        x_u = x_ref[j]
        for k in range(U):
            gs = []
            for n in range(nc):
                x_row = x_u[k: k + 1, n * LH: (n + 1) * LH]
                rhs = jnp.concatenate(
                    [hs[n], make_ext(x_row)], axis=0).astype(jnp.bfloat16)
                gs.append(jnp.dot(w_ref[...], rhs,
                                  preferred_element_type=jnp.float32))
            new_hs, new_cs = [], []
            for n in range(nc):
                h_new, c_new = cell(gs[n], cs[n])
                new_hs.append(h_new)
                new_cs.append(c_new)
            hs, cs = tuple(new_hs), tuple(new_cs)
        return (hs, cs)

    h0 = tuple(jnp.zeros((gb, LH), jnp.float32) for _ in range(nc))
    c0 = tuple(jnp.zeros((gb, LH), jnp.float32) for _ in range(nc))
    hs, cs = lax.fori_loop(0, n_outer, outer, (h0, c0), unroll=1)

    zero_row = jnp.zeros((1, LH), jnp.float32)
    for n in range(nc):
        rhs_last = jnp.concatenate(
            [hs[n], make_ext(zero_row)], axis=0).astype(jnp.bfloat16)
        out_ref[:, n * LH: (n + 1) * LH] = jnp.dot(
            fcw_ref[...], rhs_last, preferred_element_type=jnp.float32)


def kernel(x, w_ih, w_hh, b, fc_w, fc_b):
    B, T, I = x.shape
    H = w_hh.shape[-1]
    F = fc_w.shape[-1]

    GB = _round_up(H, 8)
    KD = GB + 8
    FP = _round_up(F, 8)

    NC = 8
    LH = 256
    while NC > 1 and B % (LH * NC):
        NC //= 2
    while B % LH:
        LH //= 2
    LANES = LH * NC
    assert B % LANES == 0

    U = 16
    while T % U:
        U //= 2

    def gate_rows(k, scale, units):
        rows = jnp.zeros((len(units), KD), jnp.float32)
        rows = rows.at[:, :H].set(w_hh[k].T[jnp.array(units), :] * (scale * 0.5))
        rows = rows.at[:, GB].set(w_ih[k][0][jnp.array(units)] * scale)
        rows = rows.at[:, GB + 1].set(b[k][0][jnp.array(units)] * scale)
        return rows

    main_units = list(range(48))
    blocks = [gate_rows(k, s, main_units)
              for k, s in ((0, 0.5), (1, 0.5), (3, 0.5), (2, 1.0))]
    for k, s in ((0, 0.5), (1, 0.5), (2, 1.0), (3, 0.5)):
        blocks.append(gate_rows(k, s, [48, 49]))
    w_pack = jnp.concatenate(blocks, axis=0).astype(jnp.bfloat16)

    fcw_pack = jnp.zeros((FP, KD), jnp.float32)
    fcw_pack = fcw_pack.at[:F, :H].set(fc_w.T * 0.5)
    fcw_pack = fcw_pack.at[:F, GB + 1].set(fc_b[0])
    fcw_pack = fcw_pack.astype(jnp.bfloat16)

    x_tm = jnp.transpose(x[:, :, 0]).reshape(T // U, U, B)

    grid = (B // LANES,)
    out_t = pl.pallas_call(
        functools.partial(_lstm_tp_kernel, gb=GB, nc=NC),
        out_shape=jax.ShapeDtypeStruct((FP, B), jnp.float32),
        grid=grid,
        in_specs=[
            pl.BlockSpec((T // U, U, LANES), lambda i: (0, 0, i)),
            pl.BlockSpec((200, KD), lambda i: (0, 0)),
            pl.BlockSpec((FP, KD), lambda i: (0, 0)),
        ],
        out_specs=pl.BlockSpec((FP, LANES), lambda i: (0, i)),
        compiler_params=pltpu.CompilerParams(
            dimension_semantics=("parallel",)),
    )(x_tm, w_pack, fcw_pack)

    return jnp.transpose(out_t)[:, :F]


# speedup vs baseline: 1.0804x; 1.0010x over previous
import functools

import jax
import jax.numpy as jnp
from jax import lax
from jax.experimental import pallas as pl
from jax.experimental.pallas import tpu as pltpu


def _round_up(n, m):
    return ((n + m - 1) // m) * m


def _lstm_tp_kernel(x_ref, w_ref, fcw_ref, out_ref, *, gb, nc):
    n_outer, U, L = x_ref.shape
    LH = L // nc

    row_iota = lax.broadcasted_iota(jnp.int32, (8, LH), 0)
    is_one_row = row_iota == 1

    def make_ext(x_row):
        xb = jnp.broadcast_to(x_row, (8, LH))
        return jnp.where(is_one_row, 1.0, xb)

    def cell(g, c):
        th = jnp.tanh(g[:144])
        tg = jnp.tanh(g[144:192])
        tl = jnp.tanh(g[192:200])
        ti = th[:48]
        tf = th[48:96]
        to = th[96:144]
        c_main = c[:48]
        c_tail = c[48:56]
        cm = 0.5 * ((c_main + tf * c_main) + (tg + ti * tg))
        t2m = jnp.tanh(cm)
        h2m = t2m + to * t2m
        r2 = pltpu.roll(tl, 6, 0)
        r4 = pltpu.roll(tl, 4, 0)
        r6 = pltpu.roll(tl, 2, 0)
        ctl = 0.5 * ((c_tail + r2 * c_tail) + (r4 + tl * r4))
        t2t = jnp.tanh(ctl)
        h2t = t2t + r6 * t2t
        h2 = jnp.concatenate([h2m, h2t], axis=0)
        c_new = jnp.concatenate([cm, ctl], axis=0)
        return h2, c_new

    def outer(j, carry):
        hs, cs = carry
        x_u = x_ref[j]
        for k in range(U):
            gs = []
            for n in range(nc):
                x_row = x_u[k: k + 1, n * LH: (n + 1) * LH]
                rhs = jnp.concatenate(
                    [hs[n], make_ext(x_row)], axis=0).astype(jnp.bfloat16)
                gs.append(jnp.dot(w_ref[...], rhs,
                                  preferred_element_type=jnp.float32))
            new_hs, new_cs = [], []
            for n in range(nc):
                h_new, c_new = cell(gs[n], cs[n])
                new_hs.append(h_new)
                new_cs.append(c_new)
            hs, cs = tuple(new_hs), tuple(new_cs)
        return (hs, cs)

    h0 = tuple(jnp.zeros((gb, LH), jnp.float32) for _ in range(nc))
    c0 = tuple(jnp.zeros((gb, LH), jnp.float32) for _ in range(nc))
    hs, cs = lax.fori_loop(0, n_outer, outer, (h0, c0), unroll=1)

    zero_row = jnp.zeros((1, LH), jnp.float32)
    for n in range(nc):
        rhs_last = jnp.concatenate(
            [hs[n], make_ext(zero_row)], axis=0).astype(jnp.bfloat16)
        out_ref[:, n * LH: (n + 1) * LH] = jnp.dot(
            fcw_ref[...], rhs_last, preferred_element_type=jnp.float32)


def kernel(x, w_ih, w_hh, b, fc_w, fc_b):
    B, T, I = x.shape
    H = w_hh.shape[-1]
    F = fc_w.shape[-1]

    GB = _round_up(H, 8)
    KD = GB + 8
    FP = _round_up(F, 8)

    NC = 4
    LH = 512
    while NC > 1 and B % (LH * NC):
        NC //= 2
    while B % LH:
        LH //= 2
    LANES = LH * NC
    assert B % LANES == 0

    U = 16
    while T % U:
        U //= 2

    def gate_rows(k, scale, units):
        rows = jnp.zeros((len(units), KD), jnp.float32)
        rows = rows.at[:, :H].set(w_hh[k].T[jnp.array(units), :] * (scale * 0.5))
        rows = rows.at[:, GB].set(w_ih[k][0][jnp.array(units)] * scale)
        rows = rows.at[:, GB + 1].set(b[k][0][jnp.array(units)] * scale)
        return rows

    main_units = list(range(48))
    blocks = [gate_rows(k, s, main_units)
              for k, s in ((0, 0.5), (1, 0.5), (3, 0.5), (2, 1.0))]
    for k, s in ((0, 0.5), (1, 0.5), (2, 1.0), (3, 0.5)):
        blocks.append(gate_rows(k, s, [48, 49]))
    w_pack = jnp.concatenate(blocks, axis=0).astype(jnp.bfloat16)

    fcw_pack = jnp.zeros((FP, KD), jnp.float32)
    fcw_pack = fcw_pack.at[:F, :H].set(fc_w.T * 0.5)
    fcw_pack = fcw_pack.at[:F, GB + 1].set(fc_b[0])
    fcw_pack = fcw_pack.astype(jnp.bfloat16)

    x_tm = jnp.transpose(x[:, :, 0]).reshape(T // U, U, B)

    grid = (B // LANES,)
    out_t = pl.pallas_call(
        functools.partial(_lstm_tp_kernel, gb=GB, nc=NC),
        out_shape=jax.ShapeDtypeStruct((FP, B), jnp.float32),
        grid=grid,
        in_specs=[
            pl.BlockSpec((T // U, U, LANES), lambda i: (0, 0, i)),
            pl.BlockSpec((200, KD), lambda i: (0, 0)),
            pl.BlockSpec((FP, KD), lambda i: (0, 0)),
        ],
        out_specs=pl.BlockSpec((FP, LANES), lambda i: (0, i)),
        compiler_params=pltpu.CompilerParams(
            dimension_semantics=("parallel",)),
    )(x_tm, w_pack, fcw_pack)

    return jnp.transpose(out_t)[:, :F]


# speedup vs baseline: 1.0813x; 1.0008x over previous
import functools

import jax
import jax.numpy as jnp
from jax import lax
from jax.experimental import pallas as pl
from jax.experimental.pallas import tpu as pltpu


def _round_up(n, m):
    return ((n + m - 1) // m) * m


def _lstm_tp_kernel(x_ref, w_ref, fcw_ref, out_ref, *, gb, nc):
    n_outer, U, L = x_ref.shape
    LH = L // nc

    row_iota = lax.broadcasted_iota(jnp.int32, (8, LH), 0)
    is_one_row = row_iota == 1

    def make_ext(x_row):
        xb = jnp.broadcast_to(x_row, (8, LH))
        return jnp.where(is_one_row, 1.0, xb)

    def cell(g, c):
        th = jnp.tanh(g[:144])
        tg = jnp.tanh(g[144:192])
        tl = jnp.tanh(g[192:200])
        ti = th[:48]
        tf = th[48:96]
        to = th[96:144]
        c_main = c[:48]
        c_tail = c[48:56]
        cm = 0.5 * ((c_main + tf * c_main) + (tg + ti * tg))
        t2m = jnp.tanh(cm)
        h2m = t2m + to * t2m
        r2 = pltpu.roll(tl, 6, 0)
        r4 = pltpu.roll(tl, 4, 0)
        r6 = pltpu.roll(tl, 2, 0)
        ctl = 0.5 * ((c_tail + r2 * c_tail) + (r4 + tl * r4))
        t2t = jnp.tanh(ctl)
        h2t = t2t + r6 * t2t
        h2 = jnp.concatenate([h2m, h2t], axis=0)
        c_new = jnp.concatenate([cm, ctl], axis=0)
        return h2, c_new

    def outer(j, carry):
        hs, cs = carry
        x_u = x_ref[j]
        for k in range(U):
            gs = []
            for n in range(nc):
                x_row = x_u[k: k + 1, n * LH: (n + 1) * LH]
                rhs = jnp.concatenate(
                    [hs[n], make_ext(x_row)], axis=0).astype(jnp.bfloat16)
                gs.append(jnp.dot(w_ref[...], rhs,
                                  preferred_element_type=jnp.float32))
            new_hs, new_cs = [], []
            for n in range(nc):
                h_new, c_new = cell(gs[n], cs[n])
                new_hs.append(h_new)
                new_cs.append(c_new)
            hs, cs = tuple(new_hs), tuple(new_cs)
        return (hs, cs)

    h0 = tuple(jnp.zeros((gb, LH), jnp.float32) for _ in range(nc))
    c0 = tuple(jnp.zeros((gb, LH), jnp.float32) for _ in range(nc))
    hs, cs = lax.fori_loop(0, n_outer, outer, (h0, c0), unroll=1)

    zero_row = jnp.zeros((1, LH), jnp.float32)
    for n in range(nc):
        rhs_last = jnp.concatenate(
            [hs[n], make_ext(zero_row)], axis=0).astype(jnp.bfloat16)
        out_ref[:, n * LH: (n + 1) * LH] = jnp.dot(
            fcw_ref[...], rhs_last, preferred_element_type=jnp.float32)


def kernel(x, w_ih, w_hh, b, fc_w, fc_b):
    B, T, I = x.shape
    H = w_hh.shape[-1]
    F = fc_w.shape[-1]

    GB = _round_up(H, 8)
    KD = GB + 8
    FP = _round_up(F, 8)

    NC = 4
    LH = 512
    while NC > 1 and B % (LH * NC):
        NC //= 2
    while B % LH:
        LH //= 2
    LANES = LH * NC
    assert B % LANES == 0

    U = 16
    while T % U:
        U //= 2

    def gate_rows(k, scale, units):
        rows = jnp.zeros((len(units), KD), jnp.float32)
        rows = rows.at[:, :H].set(w_hh[k].T[jnp.array(units), :] * (scale * 0.5))
        rows = rows.at[:, GB].set(w_ih[k][0][jnp.array(units)] * scale)
        rows = rows.at[:, GB + 1].set(b[k][0][jnp.array(units)] * scale)
        return rows

    main_units = list(range(48))
    blocks = [gate_rows(k, s, main_units)
              for k, s in ((0, 0.5), (1, 0.5), (3, 0.5), (2, 1.0))]
    for k, s in ((0, 0.5), (1, 0.5), (2, 1.0), (3, 0.5)):
        blocks.append(gate_rows(k, s, [48, 49]))
    w_pack = jnp.concatenate(blocks, axis=0).astype(jnp.bfloat16)

    fcw_pack = jnp.zeros((FP, KD), jnp.float32)
    fcw_pack = fcw_pack.at[:F, :H].set(fc_w.T * 0.5)
    fcw_pack = fcw_pack.at[:F, GB + 1].set(fc_b[0])
    fcw_pack = fcw_pack.astype(jnp.bfloat16)

    x_tm = jnp.transpose(x[:, :, 0]).reshape(T // U, U, B)

    grid = (B // LANES,)
    out_t = pl.pallas_call(
        functools.partial(_lstm_tp_kernel, gb=GB, nc=NC),
        out_shape=jax.ShapeDtypeStruct((FP, B), jnp.float32),
        grid=grid,
        in_specs=[
            pl.BlockSpec((T // U, U, LANES), lambda i: (0, 0, i)),
            pl.BlockSpec((200, KD), lambda i: (0, 0)),
            pl.BlockSpec((FP, KD), lambda i: (0, 0)),
        ],
        out_specs=pl.BlockSpec((FP, LANES), lambda i: (0, i)),
        compiler_params=pltpu.CompilerParams(
            dimension_semantics=("parallel",),
            flags={"XLA_TPU_STORE_TO_LOAD_FORWARDING_WINDOW": 16384}),
    )(x_tm, w_pack, fcw_pack)

    return jnp.transpose(out_t)[:, :F]
